# baseline (speedup 1.0000x reference)
import functools

import numpy as np
import jax
import jax.numpy as jnp
from jax import lax
from jax.experimental import pallas as pl
from jax.experimental.pallas import tpu as pltpu
from jax.experimental.pallas import tpu_sc as plsc

_N_DIGITS = 20
_NUM_EMB = 1000000
_EMB_DIM = 16
_NUM_BUCKETS = 100000
_NUM_HASHES = 2
_PRIME = 1000000007
_BATCH = 4096
_N_ARY = 16

_rs = np.random.RandomState(42)
_SEQ_A = int(_rs.randint(1, _PRIME))
_SEQ_B = int(_rs.randint(0, _PRIME))
_HASH_A = [int(x) for x in _rs.randint(1, _PRIME, size=_NUM_HASHES)]
_HASH_B = [int(x) for x in _rs.randint(0, _PRIME, size=_NUM_HASHES)]

_NC, _NS = 2, 16
_NW = _NC * _NS
_ROWS_PER_W = _BATCH // _NW
_NB = 64


def _addmod(x, y):
    s = x + y
    return jnp.where(s >= _PRIME, s - _PRIME, s)


def _mulmod_const(x, a):
    acc = None
    d = x
    nbits = max(a.bit_length(), 1)
    for i in range(nbits):
        if (a >> i) & 1:
            acc = d if acc is None else _addmod(acc, d)
        if i < nbits - 1:
            d = _addmod(d, d)
    return jnp.zeros_like(x) if acc is None else acc


def _dual_mulmod_const(x, a0, a1):
    acc0 = None
    acc1 = None
    d = x
    nbits = max(a0.bit_length(), a1.bit_length(), 1)
    for i in range(nbits):
        if (a0 >> i) & 1:
            acc0 = d if acc0 is None else _addmod(acc0, d)
        if (a1 >> i) & 1:
            acc1 = d if acc1 is None else _addmod(acc1, d)
        if i < nbits - 1:
            d = _addmod(d, d)
    z = jnp.zeros_like(x)
    return (z if acc0 is None else acc0), (z if acc1 is None else acc1)


def _mod_const(x, m):
    q = (x.astype(jnp.float32) * np.float32(1.0 / m)).astype(jnp.int32)
    r = x - q * m
    r = jnp.where(r < 0, r + m, r)
    r = jnp.where(r < 0, r + m, r)
    r = jnp.where(r >= m, r - m, r)
    r = jnp.where(r >= m, r - m, r)
    return r


def _hash_body(seq_ref, idx_ref):
    s = seq_ref[...]
    states = []
    state = None
    for t in range(_N_DIGITS):
        if t == 0:
            st = _addmod(s[0], _SEQ_B)
        else:
            st = _addmod(_addmod(_mulmod_const(state, _SEQ_A), _SEQ_B), s[t])
        states.append(st)
        state = st
    cl = jnp.zeros(s.shape[1:], jnp.int32)
    for t in range(_N_DIGITS):
        cl = cl + (s[t] != 0).astype(jnp.int32)
    cl = jnp.maximum(cl, 1)
    last = states[0]
    for t in range(1, _N_DIGITS):
        last = jnp.where(cl == t + 1, states[t], last)
    for t in range(_N_DIGITS):
        pt = jnp.where(t < cl, states[t], last)
        h0, h1 = _dual_mulmod_const(pt, _HASH_A[0], _HASH_A[1])
        idx_ref[0, t] = _mod_const(_addmod(h0, _HASH_B[0]), _NUM_BUCKETS)
        idx_ref[1, t] = _mod_const(_addmod(h1, _HASH_B[1]), _NUM_BUCKETS)
        im = _mod_const(pt, _NUM_EMB)
        idx_ref[2, t] = jnp.right_shift(im, jnp.int32(6))
        idx_ref[3, t] = jnp.bitwise_and(im, jnp.int32(63))


_hash_tc = pl.pallas_call(
    _hash_body,
    out_shape=jax.ShapeDtypeStruct((4, _N_DIGITS, _NW, _ROWS_PER_W), jnp.int32),
)


def _issue_half(idx_slab, buckets_hbm, imp_hbm, rows, impbuf,
                sem_rows, sem_imp, k):
    i32 = jnp.int32
    t = lax.shift_right_logical(k, i32(1))
    hb = k & i32(1)
    h = hb * i32(_NB)
    pltpu.async_copy(
        buckets_hbm.at[idx_slab.at[t, pl.ds(h, _NB)]],
        rows.at[hb, pl.ds(i32(0), _NB)], sem_rows.at[hb])
    pltpu.async_copy(
        buckets_hbm.at[idx_slab.at[t + i32(_N_DIGITS), pl.ds(h, _NB)]],
        rows.at[hb, pl.ds(i32(_NB), _NB)], sem_rows.at[hb])
    pltpu.async_copy(
        imp_hbm.at[idx_slab.at[t + i32(2 * _N_DIGITS), pl.ds(h, _NB)]],
        impbuf.at[hb], sem_imp.at[hb])


def _compute_half(idx_slab, rows, impbuf, wcompact, trep_b, out_b, t, hb):
    i32 = jnp.int32
    lane = lax.iota(jnp.int32, 16)
    last_mask = lane == i32(15)
    zeros16 = jnp.zeros((16,), jnp.int32)
    ones16 = jnp.ones((16,), jnp.int32)
    h = hb * i32(_NB)

    for g in range(_NB // 16):
        colv = idx_slab[t + i32(3 * _N_DIGITS), pl.ds(h + g * 16, 16)]
        rowsel = jnp.int32(g * 16) + lane
        fhb = jnp.full((16,), hb, jnp.int32)
        wcompact[0, pl.ds(g * 16, 16)] = plsc.load_gather(
            impbuf, [fhb, rowsel, colv])
        wcompact[1, pl.ds(g * 16, 16)] = plsc.load_gather(
            impbuf, [fhb, rowsel, colv + jnp.int32(64)])

    @plsc.parallel_loop(jnp.int32(0), jnp.int32(_NB), jnp.int32(1), unroll=6)
    def per_id(i):
        fi = jnp.full((16,), i, jnp.int32)
        col = jnp.full((16,), i, jnp.int32) + jnp.full((16,), h, jnp.int32)
        tv = plsc.load_gather(trep_b, [lane, col])
        t0 = tv * plsc.load_gather(wcompact, [zeros16, fi])
        t1 = tv * plsc.load_gather(wcompact, [ones16, fi])
        for a in range(_N_ARY):
            v = (rows[hb, i, pl.ds(a * _EMB_DIM, _EMB_DIM)] * t0
                 + rows[hb, i + _NB, pl.ds(a * _EMB_DIM, _EMB_DIM)] * t1)
            s = plsc.cumsum(v)
            plsc.store_scatter(
                out_b, [jnp.full((16,), a, jnp.int32), col], s,
                mask=last_mask)


def _sc_body(idx_hbm, trep_hbm, imp_hbm, buckets_hbm, out_hbm,
             idx_slab, rows, impbuf, wcompact, trep_blk, outbuf,
             sem_rows, sem_imp, sem_trep, sem_out):
    i32 = jnp.int32
    wid = lax.axis_index("s") * _NC + lax.axis_index("c")
    b_base = wid * _ROWS_PER_W
    pltpu.sync_copy(idx_hbm.at[:, pl.ds(b_base, _ROWS_PER_W)], idx_slab)

    _issue_half(idx_slab, buckets_hbm, imp_hbm, rows, impbuf,
                sem_rows, sem_imp, i32(0))
    pltpu.async_copy(trep_hbm.at[i32(0), :, pl.ds(b_base, _ROWS_PER_W)],
                     trep_blk.at[i32(0)], sem_trep.at[i32(0)])
    pltpu.async_copy(trep_hbm.at[i32(1), :, pl.ds(b_base, _ROWS_PER_W)],
                     trep_blk.at[i32(1)], sem_trep.at[i32(1)])

    def body(k, carry):
        hb = k & i32(1)
        t = lax.shift_right_logical(k, i32(1))
        tb = t & i32(1)

        @pl.when(k < i32(2 * _N_DIGITS - 1))
        def _():
            _issue_half(idx_slab, buckets_hbm, imp_hbm, rows, impbuf,
                        sem_rows, sem_imp, k + i32(1))

        @pl.when((hb == i32(0)) & (t >= i32(2)))
        def _():
            pltpu.make_async_copy(
                outbuf.at[tb],
                out_hbm.at[t, :, pl.ds(b_base, _ROWS_PER_W)],
                sem_out).wait()

        pltpu.make_async_copy(
            buckets_hbm.at[pl.ds(i32(0), 2 * _NB)],
            rows.at[hb], sem_rows.at[hb]).wait()
        pltpu.make_async_copy(
            imp_hbm.at[pl.ds(i32(0), _NB)],
            impbuf.at[hb], sem_imp.at[hb]).wait()

        @pl.when(hb == i32(0))
        def _():
            pltpu.make_async_copy(
                trep_hbm.at[t, :, pl.ds(b_base, _ROWS_PER_W)],
                trep_blk.at[tb], sem_trep.at[tb]).wait()

        _compute_half(idx_slab, rows, impbuf, wcompact,
                      trep_blk.at[tb], outbuf.at[tb], t, hb)

        @pl.when(hb == i32(1))
        def _():
            pltpu.async_copy(
                outbuf.at[tb],
                out_hbm.at[t, :, pl.ds(b_base, _ROWS_PER_W)], sem_out)

        @pl.when((hb == i32(1)) & (t < i32(_N_DIGITS - 2)))
        def _():
            pltpu.async_copy(
                trep_hbm.at[t + i32(2), :, pl.ds(b_base, _ROWS_PER_W)],
                trep_blk.at[tb], sem_trep.at[tb])

        return carry

    lax.fori_loop(i32(0), i32(2 * _N_DIGITS), body, i32(0))
    for t in (_N_DIGITS - 2, _N_DIGITS - 1):
        pltpu.make_async_copy(
            outbuf.at[i32(t % 2)],
            out_hbm.at[i32(t), :, pl.ds(b_base, _ROWS_PER_W)],
            sem_out).wait()


@functools.lru_cache(maxsize=1)
def _build_sc_gather():
    mesh = plsc.VectorSubcoreMesh(
        core_axis_name="c", subcore_axis_name="s",
        num_cores=_NC, num_subcores=_NS)
    return pl.kernel(
        _sc_body,
        out_type=jax.ShapeDtypeStruct((_N_DIGITS, _N_ARY, _BATCH),
                                      jnp.float32),
        mesh=mesh,
        scratch_types=[
            pltpu.VMEM((4 * _N_DIGITS, _ROWS_PER_W), jnp.int32),
            pltpu.VMEM((2, 2 * _NB, _N_ARY * _EMB_DIM), jnp.float32),
            pltpu.VMEM((2, _NB, 128), jnp.float32),
            pltpu.VMEM((2, _NB), jnp.float32),
            pltpu.VMEM((2, _EMB_DIM, _ROWS_PER_W), jnp.float32),
            pltpu.VMEM((2, _N_ARY, _ROWS_PER_W), jnp.float32),
            pltpu.SemaphoreType.DMA((2,)),
            pltpu.SemaphoreType.DMA((2,)),
            pltpu.SemaphoreType.DMA((2,)),
            pltpu.SemaphoreType.DMA,
        ],
        compiler_params=pltpu.CompilerParams(
            needs_layout_passes=False, use_tc_tiling_on_sc=True),
    )


def kernel(input_sequence, t_representation, importance, buckets):
    seq = jnp.asarray(input_sequence, jnp.int32).T.reshape(
        _N_DIGITS, _NW, _ROWS_PER_W)
    idx = _hash_tc(seq).reshape(4 * _N_DIGITS, _BATCH)
    t_rep = jnp.transpose(jnp.asarray(t_representation, jnp.float32),
                          (1, 2, 0))
    imp_f = jnp.asarray(importance, jnp.float32)
    imp = jnp.concatenate(
        [imp_f[:, 0].reshape(_NUM_EMB // 64, 64),
         imp_f[:, 1].reshape(_NUM_EMB // 64, 64)], axis=1)
    bkt = jnp.asarray(buckets, jnp.float32)
    out_t = _build_sc_gather()(idx, t_rep, imp, bkt)
    return jnp.transpose(out_t, (2, 0, 1))


# speedup vs baseline: 1.1856x; 1.1856x over previous
import functools

import numpy as np
import jax
import jax.numpy as jnp
from jax import lax
from jax.experimental import pallas as pl
from jax.experimental.pallas import tpu as pltpu
from jax.experimental.pallas import tpu_sc as plsc

_N_DIGITS = 20
_NUM_EMB = 1000000
_EMB_DIM = 16
_NUM_BUCKETS = 100000
_NUM_HASHES = 2
_PRIME = 1000000007
_BATCH = 4096
_N_ARY = 16

_rs = np.random.RandomState(42)
_SEQ_A = int(_rs.randint(1, _PRIME))
_SEQ_B = int(_rs.randint(0, _PRIME))
_HASH_A = [int(x) for x in _rs.randint(1, _PRIME, size=_NUM_HASHES)]
_HASH_B = [int(x) for x in _rs.randint(0, _PRIME, size=_NUM_HASHES)]

_NC, _NS = 2, 16
_NW = _NC * _NS
_ROWS_PER_W = _BATCH // _NW
_NB = 64


def _addmod(x, y):
    s = x + y
    return jnp.where(s >= _PRIME, s - _PRIME, s)


def _mulmod_const(x, a):
    acc = None
    d = x
    nbits = max(a.bit_length(), 1)
    for i in range(nbits):
        if (a >> i) & 1:
            acc = d if acc is None else _addmod(acc, d)
        if i < nbits - 1:
            d = _addmod(d, d)
    return jnp.zeros_like(x) if acc is None else acc


def _dual_mulmod_const(x, a0, a1):
    acc0 = None
    acc1 = None
    d = x
    nbits = max(a0.bit_length(), a1.bit_length(), 1)
    for i in range(nbits):
        if (a0 >> i) & 1:
            acc0 = d if acc0 is None else _addmod(acc0, d)
        if (a1 >> i) & 1:
            acc1 = d if acc1 is None else _addmod(acc1, d)
        if i < nbits - 1:
            d = _addmod(d, d)
    z = jnp.zeros_like(x)
    return (z if acc0 is None else acc0), (z if acc1 is None else acc1)


def _mod_const(x, m):
    q = (x.astype(jnp.float32) * np.float32(1.0 / m)).astype(jnp.int32)
    r = x - q * m
    r = jnp.where(r < 0, r + m, r)
    r = jnp.where(r < 0, r + m, r)
    r = jnp.where(r >= m, r - m, r)
    r = jnp.where(r >= m, r - m, r)
    return r


def _hash_body(seq_ref, idx_ref):
    s = seq_ref[...]
    states = []
    state = None
    for t in range(_N_DIGITS):
        if t == 0:
            st = _addmod(s[0], _SEQ_B)
        else:
            st = _addmod(_addmod(_mulmod_const(state, _SEQ_A), _SEQ_B), s[t])
        states.append(st)
        state = st
    cl = jnp.zeros(s.shape[1:], jnp.int32)
    for t in range(_N_DIGITS):
        cl = cl + (s[t] != 0).astype(jnp.int32)
    cl = jnp.maximum(cl, 1)
    last = states[0]
    for t in range(1, _N_DIGITS):
        last = jnp.where(cl == t + 1, states[t], last)
    for t in range(_N_DIGITS):
        pt = jnp.where(t < cl, states[t], last)
        h0, h1 = _dual_mulmod_const(pt, _HASH_A[0], _HASH_A[1])
        idx_ref[0, t] = _mod_const(_addmod(h0, _HASH_B[0]), _NUM_BUCKETS)
        idx_ref[1, t] = _mod_const(_addmod(h1, _HASH_B[1]), _NUM_BUCKETS)
        im = _mod_const(pt, _NUM_EMB)
        idx_ref[2, t] = jnp.right_shift(im, jnp.int32(6))
        idx_ref[3, t] = jnp.bitwise_and(im, jnp.int32(63))


_hash_tc = pl.pallas_call(
    _hash_body,
    out_shape=jax.ShapeDtypeStruct((4, _N_DIGITS, _NW, _ROWS_PER_W), jnp.int32),
)


def _issue_half(idx_slab, buckets_hbm, imp_hbm, rows, impbuf,
                sem_rows, sem_imp, k):
    i32 = jnp.int32
    t = lax.shift_right_logical(k, i32(1))
    hb = k & i32(1)
    h = hb * i32(_NB)
    pltpu.async_copy(
        buckets_hbm.at[idx_slab.at[t, pl.ds(h, _NB)]],
        rows.at[hb, pl.ds(i32(0), _NB)], sem_rows.at[hb])
    pltpu.async_copy(
        buckets_hbm.at[idx_slab.at[t + i32(_N_DIGITS), pl.ds(h, _NB)]],
        rows.at[hb, pl.ds(i32(_NB), _NB)], sem_rows.at[hb])
    pltpu.async_copy(
        imp_hbm.at[idx_slab.at[t + i32(2 * _N_DIGITS), pl.ds(h, _NB)]],
        impbuf.at[hb], sem_imp.at[hb])


def _compute_half(idx_slab, rows, impbuf, wcompact, trep_b, out_b, t, hb):
    i32 = jnp.int32
    lane = lax.iota(jnp.int32, 16)
    last_mask = lane == i32(15)
    zeros16 = jnp.zeros((16,), jnp.int32)
    ones16 = jnp.ones((16,), jnp.int32)
    h = hb * i32(_NB)

    for g in range(_NB // 16):
        colv = idx_slab[t + i32(3 * _N_DIGITS), pl.ds(h + g * 16, 16)]
        rowsel = jnp.int32(g * 16) + lane
        fhb = jnp.full((16,), hb, jnp.int32)
        wcompact[0, pl.ds(g * 16, 16)] = plsc.load_gather(
            impbuf, [fhb, rowsel, colv])
        wcompact[1, pl.ds(g * 16, 16)] = plsc.load_gather(
            impbuf, [fhb, rowsel, colv + jnp.int32(64)])

    @plsc.parallel_loop(jnp.int32(0), jnp.int32(_NB), jnp.int32(1), unroll=4)
    def per_id(i):
        fi = jnp.full((16,), i, jnp.int32)
        col = jnp.full((16,), i, jnp.int32) + jnp.full((16,), h, jnp.int32)
        tv = plsc.load_gather(trep_b, [lane, col])
        t0 = tv * plsc.load_gather(wcompact, [zeros16, fi])
        t1 = tv * plsc.load_gather(wcompact, [ones16, fi])
        for a in range(_N_ARY):
            v = (rows[hb, i, pl.ds(a * _EMB_DIM, _EMB_DIM)] * t0
                 + rows[hb, i + _NB, pl.ds(a * _EMB_DIM, _EMB_DIM)] * t1)
            s = plsc.cumsum(v)
            plsc.store_scatter(
                out_b, [jnp.full((16,), a, jnp.int32), col], s,
                mask=last_mask)


def _sc_body(idx_hbm, trep_hbm, imp_hbm, buckets_hbm, out_hbm,
             idx_slab, rows, impbuf, wcompact, trep_blk, outbuf,
             sem_rows, sem_imp, sem_trep, sem_out):
    i32 = jnp.int32
    wid = lax.axis_index("s") * _NC + lax.axis_index("c")
    b_base = wid * _ROWS_PER_W
    pltpu.sync_copy(idx_hbm.at[:, pl.ds(b_base, _ROWS_PER_W)], idx_slab)

    _issue_half(idx_slab, buckets_hbm, imp_hbm, rows, impbuf,
                sem_rows, sem_imp, i32(0))
    pltpu.async_copy(trep_hbm.at[i32(0), :, pl.ds(b_base, _ROWS_PER_W)],
                     trep_blk.at[i32(0)], sem_trep.at[i32(0)])
    pltpu.async_copy(trep_hbm.at[i32(1), :, pl.ds(b_base, _ROWS_PER_W)],
                     trep_blk.at[i32(1)], sem_trep.at[i32(1)])

    def body(k, carry):
        hb = k & i32(1)
        t = lax.shift_right_logical(k, i32(1))
        tb = t & i32(1)

        @pl.when(k < i32(2 * _N_DIGITS - 1))
        def _():
            _issue_half(idx_slab, buckets_hbm, imp_hbm, rows, impbuf,
                        sem_rows, sem_imp, k + i32(1))

        @pl.when((hb == i32(0)) & (t >= i32(2)))
        def _():
            pltpu.make_async_copy(
                outbuf.at[tb],
                out_hbm.at[t, :, pl.ds(b_base, _ROWS_PER_W)],
                sem_out).wait()

        pltpu.make_async_copy(
            buckets_hbm.at[pl.ds(i32(0), 2 * _NB)],
            rows.at[hb], sem_rows.at[hb]).wait()
        pltpu.make_async_copy(
            imp_hbm.at[pl.ds(i32(0), _NB)],
            impbuf.at[hb], sem_imp.at[hb]).wait()

        @pl.when(hb == i32(0))
        def _():
            pltpu.make_async_copy(
                trep_hbm.at[t, :, pl.ds(b_base, _ROWS_PER_W)],
                trep_blk.at[tb], sem_trep.at[tb]).wait()

        _compute_half(idx_slab, rows, impbuf, wcompact,
                      trep_blk.at[tb], outbuf.at[tb], t, hb)

        @pl.when(hb == i32(1))
        def _():
            pltpu.async_copy(
                outbuf.at[tb],
                out_hbm.at[t, :, pl.ds(b_base, _ROWS_PER_W)], sem_out)

        @pl.when((hb == i32(1)) & (t < i32(_N_DIGITS - 2)))
        def _():
            pltpu.async_copy(
                trep_hbm.at[t + i32(2), :, pl.ds(b_base, _ROWS_PER_W)],
                trep_blk.at[tb], sem_trep.at[tb])

        return carry

    lax.fori_loop(i32(0), i32(2 * _N_DIGITS), body, i32(0))
    for t in (_N_DIGITS - 2, _N_DIGITS - 1):
        pltpu.make_async_copy(
            outbuf.at[i32(t % 2)],
            out_hbm.at[i32(t), :, pl.ds(b_base, _ROWS_PER_W)],
            sem_out).wait()


@functools.lru_cache(maxsize=1)
def _build_sc_gather():
    mesh = plsc.VectorSubcoreMesh(
        core_axis_name="c", subcore_axis_name="s",
        num_cores=_NC, num_subcores=_NS)
    return pl.kernel(
        _sc_body,
        out_type=jax.ShapeDtypeStruct((_N_DIGITS, _N_ARY, _BATCH),
                                      jnp.float32),
        mesh=mesh,
        scratch_types=[
            pltpu.VMEM((4 * _N_DIGITS, _ROWS_PER_W), jnp.int32),
            pltpu.VMEM((2, 2 * _NB, _N_ARY * _EMB_DIM), jnp.float32),
            pltpu.VMEM((2, _NB, 128), jnp.float32),
            pltpu.VMEM((2, _NB), jnp.float32),
            pltpu.VMEM((2, _EMB_DIM, _ROWS_PER_W), jnp.float32),
            pltpu.VMEM((2, _N_ARY, _ROWS_PER_W), jnp.float32),
            pltpu.SemaphoreType.DMA((2,)),
            pltpu.SemaphoreType.DMA((2,)),
            pltpu.SemaphoreType.DMA((2,)),
            pltpu.SemaphoreType.DMA,
        ],
        compiler_params=pltpu.CompilerParams(
            needs_layout_passes=False, use_tc_tiling_on_sc=True),
    )


def kernel(input_sequence, t_representation, importance, buckets):
    seq = jnp.asarray(input_sequence, jnp.int32).T.reshape(
        _N_DIGITS, _NW, _ROWS_PER_W)
    idx = _hash_tc(seq).reshape(4 * _N_DIGITS, _BATCH)
    t_rep = jnp.transpose(jnp.asarray(t_representation, jnp.float32),
                          (1, 2, 0))
    imp_f = jnp.asarray(importance, jnp.float32)
    imp = jnp.concatenate(
        [imp_f[:, 0].reshape(_NUM_EMB // 64, 64),
         imp_f[:, 1].reshape(_NUM_EMB // 64, 64)], axis=1)
    bkt = jnp.asarray(buckets, jnp.float32)
    out_t = _build_sc_gather()(idx, t_rep, imp, bkt)
    return jnp.transpose(out_t, (2, 0, 1))
